# baseline (speedup 1.0000x reference)
import functools

import jax
import jax.numpy as jnp
from jax import lax
from jax.experimental import pallas as pl
from jax.experimental.pallas import tpu as pltpu
from jax.experimental.pallas import tpu_sc as plsc

N = 10000
E = 320000
D = 128

NC = 2
NS = 16
L = 16

CHK = 128
ROWS = 2560
EP = ROWS * CHK
RPTC = ROWS // NS
NB = 40
NBLK = RPTC // NB
NP = N + 8
RPT = 624
TAIL = N - NS * RPT


def _sc_body(x_hbm, e0_hbm, e1_hbm, zr_hbm, on_hbm, acc_hbm, cnt_hbm,
             acc_sh, gblk, sblk, rows0, rows1, sem0, sem1):
    c = lax.axis_index("c")
    s = lax.axis_index("s")
    r0 = s * RPT

    def zero_acc():
        pltpu.sync_copy(zr_hbm, rows0)
        for m in range(4):
            pltpu.sync_copy(rows0, acc_sh.at[pl.ds(r0 + m * CHK, CHK)])
        pltpu.sync_copy(rows0.at[pl.ds(0, RPT - 4 * CHK)],
                        acc_sh.at[pl.ds(r0 + 4 * CHK, RPT - 4 * CHK)])

        @pl.when(s == NS - 1)
        def _():
            pltpu.sync_copy(rows0.at[pl.ds(0, TAIL)],
                            acc_sh.at[pl.ds(NS * RPT, TAIL)])

    def copy_out(dst_hbm):
        for m in range(4):
            pltpu.sync_copy(acc_sh.at[pl.ds(r0 + m * CHK, CHK)], rows0)
            pltpu.sync_copy(rows0, dst_hbm.at[c, pl.ds(r0 + m * CHK, CHK)])
        w = RPT - 4 * CHK
        pltpu.sync_copy(acc_sh.at[pl.ds(r0 + 4 * CHK, w)],
                        rows0.at[pl.ds(0, w)])
        pltpu.sync_copy(rows0.at[pl.ds(0, w)],
                        dst_hbm.at[c, pl.ds(r0 + 4 * CHK, w)])

        @pl.when(s == NS - 1)
        def _():
            pltpu.sync_copy(acc_sh.at[pl.ds(NS * RPT, TAIL)],
                            rows0.at[pl.ds(0, TAIL)])
            pltpu.sync_copy(rows0.at[pl.ds(0, TAIL)],
                            dst_hbm.at[c, pl.ds(NS * RPT, TAIL)])

    def edge_loop(body):
        @pl.when(c == 0)
        def _():
            body(e0_hbm, e1_hbm)

        @pl.when(c == 1)
        def _():
            body(e1_hbm, e0_hbm)

    def drain(dst, sem):
        pltpu.make_async_copy(x_hbm.at[pl.ds(0, CHK)], dst, sem).wait()

    zero_acc()
    plsc.subcore_barrier()

    def acc_dir(g_hbm, s_hbm):
        for b in range(NBLK):
            base = s * RPTC + b * NB
            pltpu.sync_copy(g_hbm.at[pl.ds(base, NB)], gblk)
            pltpu.sync_copy(s_hbm.at[pl.ds(base, NB)], sblk)
            pltpu.async_copy(x_hbm.at[gblk.at[0]], rows0, sem0)

            def pair(m, carry):
                q = 2 * m
                pltpu.async_copy(x_hbm.at[gblk.at[q + 1]], rows1, sem1)
                drain(rows0, sem0)
                pltpu.sync_copy(rows0, acc_sh.at[sblk.at[q]], add=True)

                @pl.when(q + 2 < NB)
                def _():
                    pltpu.async_copy(x_hbm.at[gblk.at[q + 2]], rows0, sem0)

                drain(rows1, sem1)
                pltpu.sync_copy(rows1, acc_sh.at[sblk.at[q + 1]], add=True)
                return carry

            lax.fori_loop(0, NB // 2, pair, 0)

    edge_loop(acc_dir)
    plsc.subcore_barrier()
    copy_out(acc_hbm)
    plsc.subcore_barrier()

    zero_acc()
    plsc.subcore_barrier()
    pltpu.sync_copy(on_hbm, rows0)

    def cnt_dir(g_hbm, s_hbm):
        for b in range(NBLK):
            base = s * RPTC + b * NB
            pltpu.sync_copy(s_hbm.at[pl.ds(base, NB)], sblk)
            pltpu.async_copy(rows0, acc_sh.at[sblk.at[0]], sem0, add=True)

            def pair(m, carry):
                q = 2 * m
                pltpu.async_copy(rows0, acc_sh.at[sblk.at[q + 1]], sem1,
                                 add=True)
                drain(rows1, sem0)

                @pl.when(q + 2 < NB)
                def _():
                    pltpu.async_copy(rows0, acc_sh.at[sblk.at[q + 2]], sem0,
                                     add=True)

                drain(rows1, sem1)
                return carry

            lax.fori_loop(0, NB // 2, pair, 0)

    edge_loop(cnt_dir)
    plsc.subcore_barrier()
    copy_out(cnt_hbm)


@functools.cache
def _sc_aggregate():
    return pl.kernel(
        lambda *args: _sc_body(*args),
        out_type=(jax.ShapeDtypeStruct((NC, N, D), jnp.float32),
                  jax.ShapeDtypeStruct((NC, N, D), jnp.float32)),
        mesh=plsc.VectorSubcoreMesh(core_axis_name="c", subcore_axis_name="s",
                                    num_cores=NC, num_subcores=NS),
        scratch_types=[
            pltpu.VMEM_SHARED((NP, D), jnp.float32),
            pltpu.VMEM((NB, CHK), jnp.int32),
            pltpu.VMEM((NB, CHK), jnp.int32),
            pltpu.VMEM((CHK, D), jnp.float32),
            pltpu.VMEM((CHK, D), jnp.float32),
            pltpu.SemaphoreType.DMA,
            pltpu.SemaphoreType.DMA,
        ],
    )


R = 400


def _tc_body(acc_i, cnt_i, acc_o, cnt_o, x_ref,
             wli, wlo, wri, wro, bli, blo, out_ref):
    mi = acc_i[...] / jnp.maximum(cnt_i[:, 0:1], 1.0)
    mo = acc_o[...] / jnp.maximum(cnt_o[:, 0:1], 1.0)
    o = jnp.dot(mi, wli[...], preferred_element_type=jnp.float32)
    o = o + jnp.dot(mo, wlo[...], preferred_element_type=jnp.float32)
    o = o + jnp.dot(x_ref[...], wri[...] + wro[...],
                    preferred_element_type=jnp.float32)
    out_ref[...] = o + bli[0:1, :] + blo[0:1, :]


def _tc_combine(acc_in, cnt_in, acc_out, cnt_out, x,
                wli_t, wlo_t, wri_t, wro_t, bli, blo):
    blk = lambda w: pl.BlockSpec((R, w), lambda i: (i, 0))
    full = pl.BlockSpec((D, D), lambda i: (0, 0))
    bias = pl.BlockSpec((8, D), lambda i: (0, 0))
    return pl.pallas_call(
        _tc_body,
        grid=(N // R,),
        in_specs=[blk(D), blk(D), blk(D), blk(D), blk(D),
                  full, full, full, full, bias, bias],
        out_specs=blk(D),
        out_shape=jax.ShapeDtypeStruct((N, D), jnp.float32),
    )(acc_in, cnt_in, acc_out, cnt_out, x,
      wli_t, wlo_t, wri_t, wro_t, bli, blo)


def kernel(x, ei, Wl_in, bl_in, Wr_in, Wl_out, bl_out, Wr_out):
    zr = jnp.zeros((CHK, D), jnp.float32)
    on = jnp.ones((CHK, D), jnp.float32)
    x_pad = jnp.concatenate([x, jnp.zeros((NP - N, D), jnp.float32)], axis=0)
    pad = jnp.full((EP - E,), N, jnp.int32)
    e0r = jnp.concatenate([ei[0], pad]).reshape(ROWS, CHK)
    e1r = jnp.concatenate([ei[1], pad]).reshape(ROWS, CHK)
    acc, cnt = _sc_aggregate()(x_pad, e0r, e1r, zr, on)
    return _tc_combine(
        acc[0], cnt[0], acc[1], cnt[1], x,
        Wl_in.T, Wl_out.T, Wr_in.T, Wr_out.T,
        jnp.broadcast_to(bl_in.reshape(1, D), (8, D)),
        jnp.broadcast_to(bl_out.reshape(1, D), (8, D)))


# speedup vs baseline: 1.2143x; 1.2143x over previous
import functools

import jax
import jax.numpy as jnp
from jax import lax
from jax.experimental import pallas as pl
from jax.experimental.pallas import tpu as pltpu
from jax.experimental.pallas import tpu_sc as plsc

N = 10000
E = 320000
D = 128

NC = 2
NS = 16
L = 16

EPW = E // NS
K = 80
NCHUNK = EPW // K
RPT = 624
TAIL = N - NS * RPT


def _sc_body(x_hbm, e0_hbm, e1_hbm, zr_hbm, on_hbm, acc_hbm, cnt_hbm,
             acc_sh, gidx, sidx, rows, ones, sem, sem2):
    c = lax.axis_index("c")
    s = lax.axis_index("s")
    r0 = s * RPT

    def zero_acc():
        pltpu.sync_copy(zr_hbm, rows)
        for m in range(7):
            pltpu.sync_copy(rows, acc_sh.at[pl.ds(r0 + m * K, K)])
        pltpu.sync_copy(rows.at[pl.ds(0, RPT - 7 * K)],
                        acc_sh.at[pl.ds(r0 + 7 * K, RPT - 7 * K)])

        @pl.when(s == NS - 1)
        def _():
            pltpu.sync_copy(rows.at[pl.ds(0, TAIL)],
                            acc_sh.at[pl.ds(NS * RPT, TAIL)])

    def copy_out(dst_hbm):
        for m in range(7):
            pltpu.sync_copy(acc_sh.at[pl.ds(r0 + m * K, K)], rows)
            pltpu.sync_copy(rows, dst_hbm.at[c, pl.ds(r0 + m * K, K)])
        w = RPT - 7 * K
        pltpu.sync_copy(acc_sh.at[pl.ds(r0 + 7 * K, w)], rows.at[pl.ds(0, w)])
        pltpu.sync_copy(rows.at[pl.ds(0, w)],
                        dst_hbm.at[c, pl.ds(r0 + 7 * K, w)])

        @pl.when(s == NS - 1)
        def _():
            pltpu.sync_copy(acc_sh.at[pl.ds(NS * RPT, TAIL)],
                            rows.at[pl.ds(0, TAIL)])
            pltpu.sync_copy(rows.at[pl.ds(0, TAIL)],
                            dst_hbm.at[c, pl.ds(NS * RPT, TAIL)])

    def edge_loop(body):
        @pl.when(c == 0)
        def _():
            body(e0_hbm, e1_hbm)

        @pl.when(c == 1)
        def _():
            body(e1_hbm, e0_hbm)

    zero_acc()
    plsc.subcore_barrier()

    def acc_dir(g_hbm, s_hbm):
        def step(j, carry):
            base = s * EPW + j * K
            pltpu.sync_copy(g_hbm.at[pl.ds(base, K)], gidx)
            pltpu.sync_copy(s_hbm.at[pl.ds(base, K)], sidx)
            pltpu.async_copy(x_hbm.at[gidx], rows, sem).wait()
            pltpu.sync_copy(rows, acc_sh.at[sidx], add=True)
            return carry
        lax.fori_loop(0, NCHUNK, step, 0)

    edge_loop(acc_dir)
    plsc.subcore_barrier()
    copy_out(acc_hbm)
    plsc.subcore_barrier()

    zero_acc()
    pltpu.sync_copy(on_hbm, ones)
    plsc.subcore_barrier()

    def cnt_dir(g_hbm, s_hbm):
        def drain(sem):
            pltpu.make_async_copy(zr_hbm, ones, sem).wait()

        pltpu.sync_copy(s_hbm.at[pl.ds(s * EPW, K)], gidx)
        pltpu.async_copy(ones, acc_sh.at[gidx], sem, add=True)

        def pair(m, carry):
            q = 2 * m
            base = s * EPW + (q + 1) * K
            pltpu.sync_copy(s_hbm.at[pl.ds(base, K)], sidx)
            pltpu.async_copy(ones, acc_sh.at[sidx], sem2, add=True)
            drain(sem)

            @pl.when(q + 2 < NCHUNK)
            def _():
                base2 = s * EPW + (q + 2) * K
                pltpu.sync_copy(s_hbm.at[pl.ds(base2, K)], gidx)
                pltpu.async_copy(ones, acc_sh.at[gidx], sem, add=True)

            drain(sem2)
            return carry

        lax.fori_loop(0, NCHUNK // 2, pair, 0)

    edge_loop(cnt_dir)
    plsc.subcore_barrier()
    copy_out(cnt_hbm)


@functools.cache
def _sc_aggregate():
    return pl.kernel(
        lambda *args: _sc_body(*args),
        out_type=(jax.ShapeDtypeStruct((NC, N, D), jnp.float32),
                  jax.ShapeDtypeStruct((NC, N, D), jnp.float32)),
        mesh=plsc.VectorSubcoreMesh(core_axis_name="c", subcore_axis_name="s",
                                    num_cores=NC, num_subcores=NS),
        scratch_types=[
            pltpu.VMEM_SHARED((N, D), jnp.float32),
            pltpu.VMEM((K,), jnp.int32),
            pltpu.VMEM((K,), jnp.int32),
            pltpu.VMEM((K, D), jnp.float32),
            pltpu.VMEM((K, D), jnp.float32),
            pltpu.SemaphoreType.DMA,
            pltpu.SemaphoreType.DMA,
        ],
    )


R = 400


def _tc_body(acc_i, cnt_i, acc_o, cnt_o, x_ref,
             wli, wlo, wri, wro, bli, blo, out_ref):
    mi = acc_i[...] / jnp.maximum(cnt_i[:, 0:1], 1.0)
    mo = acc_o[...] / jnp.maximum(cnt_o[:, 0:1], 1.0)
    o = jnp.dot(mi, wli[...], preferred_element_type=jnp.float32)
    o = o + jnp.dot(mo, wlo[...], preferred_element_type=jnp.float32)
    o = o + jnp.dot(x_ref[...], wri[...] + wro[...],
                    preferred_element_type=jnp.float32)
    out_ref[...] = o + bli[0:1, :] + blo[0:1, :]


def _tc_combine(acc_in, cnt_in, acc_out, cnt_out, x,
                wli_t, wlo_t, wri_t, wro_t, bli, blo):
    blk = lambda w: pl.BlockSpec((R, w), lambda i: (i, 0))
    full = pl.BlockSpec((D, D), lambda i: (0, 0))
    bias = pl.BlockSpec((8, D), lambda i: (0, 0))
    return pl.pallas_call(
        _tc_body,
        grid=(N // R,),
        in_specs=[blk(D), blk(D), blk(D), blk(D), blk(D),
                  full, full, full, full, bias, bias],
        out_specs=blk(D),
        out_shape=jax.ShapeDtypeStruct((N, D), jnp.float32),
    )(acc_in, cnt_in, acc_out, cnt_out, x,
      wli_t, wlo_t, wri_t, wro_t, bli, blo)


def kernel(x, ei, Wl_in, bl_in, Wr_in, Wl_out, bl_out, Wr_out):
    zr = jnp.zeros((K, D), jnp.float32)
    on = jnp.ones((K, D), jnp.float32)
    acc, cnt = _sc_aggregate()(x, ei[0], ei[1], zr, on)
    return _tc_combine(
        acc[0], cnt[0], acc[1], cnt[1], x,
        Wl_in.T, Wl_out.T, Wr_in.T, Wr_out.T,
        jnp.broadcast_to(bl_in.reshape(1, D), (8, D)),
        jnp.broadcast_to(bl_out.reshape(1, D), (8, D)))


# speedup vs baseline: 1.3932x; 1.1473x over previous
import functools

import jax
import jax.numpy as jnp
from jax import lax
from jax.experimental import pallas as pl
from jax.experimental.pallas import tpu as pltpu
from jax.experimental.pallas import tpu_sc as plsc

N = 10000
E = 320000
D = 128

NC = 2
NS = 16
L = 16

EPW = E // NS
K = 80
NCHUNK = EPW // K
RPT = 624
TAIL = N - NS * RPT


def _sc_body(x_hbm, e0_hbm, e1_hbm, zr_hbm, on_hbm, acc_hbm, cnt_hbm,
             acc_sh, gidx, sidx, sidx2, rows, rows2, ones, sem, sem2, sem3):
    c = lax.axis_index("c")
    s = lax.axis_index("s")
    r0 = s * RPT

    def zero_acc():
        pltpu.sync_copy(zr_hbm, rows)
        for m in range(7):
            pltpu.sync_copy(rows, acc_sh.at[pl.ds(r0 + m * K, K)])
        pltpu.sync_copy(rows.at[pl.ds(0, RPT - 7 * K)],
                        acc_sh.at[pl.ds(r0 + 7 * K, RPT - 7 * K)])

        @pl.when(s == NS - 1)
        def _():
            pltpu.sync_copy(rows.at[pl.ds(0, TAIL)],
                            acc_sh.at[pl.ds(NS * RPT, TAIL)])

    def copy_out(dst_hbm):
        for m in range(7):
            pltpu.sync_copy(acc_sh.at[pl.ds(r0 + m * K, K)], rows)
            pltpu.sync_copy(rows, dst_hbm.at[c, pl.ds(r0 + m * K, K)])
        w = RPT - 7 * K
        pltpu.sync_copy(acc_sh.at[pl.ds(r0 + 7 * K, w)], rows.at[pl.ds(0, w)])
        pltpu.sync_copy(rows.at[pl.ds(0, w)],
                        dst_hbm.at[c, pl.ds(r0 + 7 * K, w)])

        @pl.when(s == NS - 1)
        def _():
            pltpu.sync_copy(acc_sh.at[pl.ds(NS * RPT, TAIL)],
                            rows.at[pl.ds(0, TAIL)])
            pltpu.sync_copy(rows.at[pl.ds(0, TAIL)],
                            dst_hbm.at[c, pl.ds(NS * RPT, TAIL)])

    def edge_loop(body):
        @pl.when(c == 0)
        def _():
            body(e0_hbm, e1_hbm)

        @pl.when(c == 1)
        def _():
            body(e1_hbm, e0_hbm)

    zero_acc()
    plsc.subcore_barrier()

    def acc_dir(g_hbm, s_hbm):
        def drain(sem_):
            pltpu.make_async_copy(zr_hbm, rows, sem_).wait()

        def fetch(j, idx_ref, buf_ref):
            base = s * EPW + j * K
            pltpu.sync_copy(g_hbm.at[pl.ds(base, K)], gidx)
            pltpu.sync_copy(s_hbm.at[pl.ds(base, K)], idx_ref)
            pltpu.async_copy(x_hbm.at[gidx], buf_ref, sem).wait()

        fetch(0, sidx, rows)

        def pair(m, carry):
            q = 2 * m
            pltpu.async_copy(rows, acc_sh.at[sidx], sem2, add=True)
            fetch(q + 1, sidx2, rows2)
            pltpu.async_copy(rows2, acc_sh.at[sidx2], sem3, add=True)
            drain(sem2)

            @pl.when(q + 2 < NCHUNK)
            def _():
                fetch(q + 2, sidx, rows)

            drain(sem3)
            return carry

        lax.fori_loop(0, NCHUNK // 2, pair, 0)

    edge_loop(acc_dir)
    plsc.subcore_barrier()
    copy_out(acc_hbm)
    plsc.subcore_barrier()

    zero_acc()
    pltpu.sync_copy(on_hbm, ones)
    plsc.subcore_barrier()

    def cnt_dir(g_hbm, s_hbm):
        def drain(sem):
            pltpu.make_async_copy(zr_hbm, ones, sem).wait()

        pltpu.sync_copy(s_hbm.at[pl.ds(s * EPW, K)], gidx)
        pltpu.async_copy(ones, acc_sh.at[gidx], sem, add=True)

        def pair(m, carry):
            q = 2 * m
            base = s * EPW + (q + 1) * K
            pltpu.sync_copy(s_hbm.at[pl.ds(base, K)], sidx)
            pltpu.async_copy(ones, acc_sh.at[sidx], sem2, add=True)
            drain(sem)

            @pl.when(q + 2 < NCHUNK)
            def _():
                base2 = s * EPW + (q + 2) * K
                pltpu.sync_copy(s_hbm.at[pl.ds(base2, K)], gidx)
                pltpu.async_copy(ones, acc_sh.at[gidx], sem, add=True)

            drain(sem2)
            return carry

        lax.fori_loop(0, NCHUNK // 2, pair, 0)

    edge_loop(cnt_dir)
    plsc.subcore_barrier()
    copy_out(cnt_hbm)


@functools.cache
def _sc_aggregate():
    return pl.kernel(
        lambda *args: _sc_body(*args),
        out_type=(jax.ShapeDtypeStruct((NC, N, D), jnp.float32),
                  jax.ShapeDtypeStruct((NC, N, D), jnp.float32)),
        mesh=plsc.VectorSubcoreMesh(core_axis_name="c", subcore_axis_name="s",
                                    num_cores=NC, num_subcores=NS),
        scratch_types=[
            pltpu.VMEM_SHARED((N, D), jnp.float32),
            pltpu.VMEM((K,), jnp.int32),
            pltpu.VMEM((K,), jnp.int32),
            pltpu.VMEM((K,), jnp.int32),
            pltpu.VMEM((K, D), jnp.float32),
            pltpu.VMEM((K, D), jnp.float32),
            pltpu.VMEM((K, D), jnp.float32),
            pltpu.SemaphoreType.DMA,
            pltpu.SemaphoreType.DMA,
            pltpu.SemaphoreType.DMA,
        ],
    )


R = 400


def _tc_body(acc_i, cnt_i, acc_o, cnt_o, x_ref,
             wli, wlo, wri, wro, bli, blo, out_ref):
    mi = acc_i[...] / jnp.maximum(cnt_i[:, 0:1], 1.0)
    mo = acc_o[...] / jnp.maximum(cnt_o[:, 0:1], 1.0)
    o = jnp.dot(mi, wli[...], preferred_element_type=jnp.float32)
    o = o + jnp.dot(mo, wlo[...], preferred_element_type=jnp.float32)
    o = o + jnp.dot(x_ref[...], wri[...] + wro[...],
                    preferred_element_type=jnp.float32)
    out_ref[...] = o + bli[0:1, :] + blo[0:1, :]


def _tc_combine(acc_in, cnt_in, acc_out, cnt_out, x,
                wli_t, wlo_t, wri_t, wro_t, bli, blo):
    blk = lambda w: pl.BlockSpec((R, w), lambda i: (i, 0))
    full = pl.BlockSpec((D, D), lambda i: (0, 0))
    bias = pl.BlockSpec((8, D), lambda i: (0, 0))
    return pl.pallas_call(
        _tc_body,
        grid=(N // R,),
        in_specs=[blk(D), blk(D), blk(D), blk(D), blk(D),
                  full, full, full, full, bias, bias],
        out_specs=blk(D),
        out_shape=jax.ShapeDtypeStruct((N, D), jnp.float32),
    )(acc_in, cnt_in, acc_out, cnt_out, x,
      wli_t, wlo_t, wri_t, wro_t, bli, blo)


def kernel(x, ei, Wl_in, bl_in, Wr_in, Wl_out, bl_out, Wr_out):
    zr = jnp.zeros((K, D), jnp.float32)
    on = jnp.ones((K, D), jnp.float32)
    acc, cnt = _sc_aggregate()(x, ei[0], ei[1], zr, on)
    return _tc_combine(
        acc[0], cnt[0], acc[1], cnt[1], x,
        Wl_in.T, Wl_out.T, Wr_in.T, Wr_out.T,
        jnp.broadcast_to(bl_in.reshape(1, D), (8, D)),
        jnp.broadcast_to(bl_out.reshape(1, D), (8, D)))
